# baseline (speedup 1.0000x reference)
import functools

import jax
import jax.numpy as jnp
from jax import lax
from jax.experimental import pallas as pl
from jax.experimental.pallas import tpu as pltpu
from jax.experimental.pallas import tpu_sc as plsc

BATCH = 16384
FIELDS = 26
DIM = 128
NUM_CORES = 2
NUM_SUBCORES = 16
NUM_WORKERS = NUM_CORES * NUM_SUBCORES
BATCH_PER_WORKER = BATCH // NUM_WORKERS
GB = 16
NGROUPS = BATCH_PER_WORKER // GB

_mesh = plsc.VectorSubcoreMesh(core_axis_name="c", subcore_axis_name="s")


@functools.partial(
    pl.kernel,
    mesh=_mesh,
    compiler_params=pltpu.CompilerParams(use_tc_tiling_on_sc=False),
    out_type=jax.ShapeDtypeStruct((BATCH, FIELDS, DIM), jnp.float32),
    scratch_types=[
        pltpu.VMEM((BATCH_PER_WORKER, FIELDS), jnp.int32),
        pltpu.VMEM((GB, FIELDS, DIM), jnp.float32),
        pltpu.VMEM((GB, FIELDS, DIM), jnp.float32),
        pltpu.SemaphoreType.DMA,
        pltpu.SemaphoreType.DMA,
        pltpu.SemaphoreType.DMA,
        pltpu.SemaphoreType.DMA,
    ],
)
def _sc_gather(idx_hbm, table_hbm, out_hbm, idx_v, rows0, rows1,
               gsem0, gsem1, wsem0, wsem1):
    wid = lax.axis_index("s") * NUM_CORES + lax.axis_index("c")
    batch_base = wid * BATCH_PER_WORKER
    rows = (rows0, rows1)
    gsem = (gsem0, gsem1)
    wsem = (wsem0, wsem1)

    pltpu.sync_copy(idx_hbm.at[pl.ds(batch_base, BATCH_PER_WORKER)], idx_v)

    def start_gathers(group, s):
        for b in range(GB):
            bb = group * GB + b
            pltpu.async_copy(table_hbm.at[idx_v.at[bb]], rows[s].at[b], gsem[s])

    def wait_gathers(s):
        for b in range(GB):
            pltpu.make_async_copy(
                out_hbm.at[0], rows[s].at[b], gsem[s]
            ).wait()

    def start_writes(group, s):
        pltpu.async_copy(
            rows[s], out_hbm.at[pl.ds(batch_base + group * GB, GB)], wsem[s]
        )

    def wait_writes(s):
        pltpu.make_async_copy(
            rows[s], out_hbm.at[pl.ds(0, GB)], wsem[s]
        ).wait()

    start_gathers(0, 0)

    def body(i, carry):
        h0 = 2 * i
        @pl.when(h0 >= 1)
        def _():
            wait_writes(1)

        start_gathers(h0 + 1, 1)
        wait_gathers(0)
        start_writes(h0, 0)
        wait_writes(0)

        @pl.when(h0 + 2 < NGROUPS)
        def _():
            start_gathers(h0 + 2, 0)

        wait_gathers(1)
        start_writes(h0 + 1, 1)
        return carry

    lax.fori_loop(0, NGROUPS // 2, body, 0)
    wait_writes(1)


def kernel(x, weight):
    return _sc_gather(x, weight)


# speedup vs baseline: 3.5516x; 3.5516x over previous
import functools

import jax
import jax.numpy as jnp
from jax import lax
from jax.experimental import pallas as pl
from jax.experimental.pallas import tpu as pltpu
from jax.experimental.pallas import tpu_sc as plsc

BATCH = 16384
FIELDS = 26
DIM = 128
TOTAL_ROWS = BATCH * FIELDS
NUM_CORES = 2
NUM_SUBCORES = 16
NUM_WORKERS = NUM_CORES * NUM_SUBCORES
ROWS_PER_WORKER = TOTAL_ROWS // NUM_WORKERS
GATHER_ROWS = 128
BLOCKS_PER_WORKER = ROWS_PER_WORKER // GATHER_ROWS
IDX_BLOCKS = TOTAL_ROWS // GATHER_ROWS
NB = 2
NGROUPS = BLOCKS_PER_WORKER // NB

_mesh = plsc.VectorSubcoreMesh(core_axis_name="c", subcore_axis_name="s")


@functools.partial(
    pl.kernel,
    mesh=_mesh,
    compiler_params=pltpu.CompilerParams(use_tc_tiling_on_sc=False),
    out_type=jax.ShapeDtypeStruct((TOTAL_ROWS, DIM), jnp.float32),
    scratch_types=[
        pltpu.VMEM((BLOCKS_PER_WORKER, GATHER_ROWS), jnp.int32),
        pltpu.VMEM((NB, GATHER_ROWS, DIM), jnp.float32),
        pltpu.VMEM((NB, GATHER_ROWS, DIM), jnp.float32),
        pltpu.SemaphoreType.DMA,
        pltpu.SemaphoreType.DMA,
        pltpu.SemaphoreType.DMA,
        pltpu.SemaphoreType.DMA,
    ],
)
def _sc_gather(idx_hbm, table_hbm, out_hbm, idx_v, rows0, rows1,
               gsem0, gsem1, wsem0, wsem1):
    wid = lax.axis_index("s") * NUM_CORES + lax.axis_index("c")
    blk_base = wid * BLOCKS_PER_WORKER
    rows = (rows0, rows1)
    gsem = (gsem0, gsem1)
    wsem = (wsem0, wsem1)

    pltpu.sync_copy(idx_hbm.at[pl.ds(blk_base, BLOCKS_PER_WORKER)], idx_v)

    def start_gathers(group, s):
        for b in range(NB):
            blk = group * NB + b
            pltpu.async_copy(table_hbm.at[idx_v.at[blk]], rows[s].at[b], gsem[s])

    def wait_gathers(s):
        for b in range(NB):
            pltpu.make_async_copy(
                out_hbm.at[pl.ds(0, GATHER_ROWS)], rows[s].at[b], gsem[s]
            ).wait()

    def start_writes(group, s):
        for b in range(NB):
            blk = group * NB + b
            pltpu.async_copy(
                rows[s].at[b],
                out_hbm.at[pl.ds((blk_base + blk) * GATHER_ROWS, GATHER_ROWS)],
                wsem[s],
            )

    def wait_writes(s):
        for b in range(NB):
            pltpu.make_async_copy(
                rows[s].at[b], out_hbm.at[pl.ds(0, GATHER_ROWS)], wsem[s]
            ).wait()

    start_gathers(0, 0)

    def body(i, carry):
        h0 = 2 * i
        @pl.when(h0 >= 1)
        def _():
            wait_writes(1)

        start_gathers(h0 + 1, 1)
        wait_gathers(0)
        start_writes(h0, 0)
        wait_writes(0)

        @pl.when(h0 + 2 < NGROUPS)
        def _():
            start_gathers(h0 + 2, 0)

        wait_gathers(1)
        start_writes(h0 + 1, 1)
        return carry

    lax.fori_loop(0, NGROUPS // 2, body, 0)
    wait_writes(1)


def kernel(x, weight):
    idx = x.T.reshape(IDX_BLOCKS, GATHER_ROWS)
    out = _sc_gather(idx, weight)
    return out.reshape(FIELDS, BATCH, DIM).transpose(1, 0, 2)
